# Initial kernel scaffold; baseline (speedup 1.0000x reference)
import jax
import jax.numpy as jnp
from jax.experimental import pallas as pl


def kernel(batch, emb_table, fc_w, fc_b):
    raise NotImplementedError("write your pallas kernel here")


# speedup vs baseline: 2.8348x; 2.8348x over previous
import functools

import jax
import jax.numpy as jnp
from jax import lax
from jax.experimental import pallas as pl
from jax.experimental.pallas import tpu as pltpu
from jax.experimental.pallas import tpu_sc as plsc

_VOCAB = 100000
_MAX_LEN = 200
_EMBED_DIM = 100
_NUM_CLASSES = 5
_BATCH = 4096

_NC = 2
_NS = 16
_NW = _NC * _NS
_B_PER_TILE = _BATCH // _NW
_B_BLK = 4
_N_GROUPS = _B_PER_TILE // _B_BLK
_L_CHUNK = 100
_N_CHUNK = _MAX_LEN // _L_CHUNK
_DPAD = 112
_KV = _DPAD // 16
_NGRP = 4


def _sc_body(table, batchr, wpk, out, w_v, rows_v, idx_v, out_v, sem):
    cid = lax.axis_index("c")
    sid = lax.axis_index("s")
    wid = sid * _NC + cid
    b0 = wid * _B_PER_TILE

    pltpu.sync_copy(wpk, w_v)

    def group_body(g, carry):
        for j in range(_B_BLK):
            pltpu.sync_copy(batchr.at[b0 + g * _B_BLK + j], idx_v.at[j])

        acc = [jnp.zeros((16,), jnp.float32)] * (_B_BLK * _NUM_CLASSES)

        for ci in range(_N_CHUNK):
            cps = [
                pltpu.async_copy(table.at[idx_v.at[j, ci]], rows_v.at[j], sem)
                for j in range(_B_BLK)
            ]
            for cp in cps:
                cp.wait()

            def l_body(l, acc, ci=ci):
                acc = list(acc)
                lg = ci * _L_CHUNK + l
                for kh in (range(0, 4), range(4, _KV)):
                    xs = [
                        [rows_v[j, l, pl.ds(k * 16, 16)] for k in kh]
                        for j in range(_B_BLK)
                    ]
                    for c in range(_NUM_CLASSES):
                        ws = []
                        unpacked = {}
                        for k in kh:
                            gq, half = divmod(k, 2)
                            if gq not in unpacked:
                                wword = w_v[c, lg, gq, :]
                                unpacked[gq] = plsc.unpack(
                                    wword,
                                    format=plsc.PackFormat.INTERLEAVED,
                                    preferred_element_type=jnp.float32,
                                )
                            ws.append(unpacked[gq][half])
                        for j in range(_B_BLK):
                            for i, k in enumerate(kh):
                                acc[j * _NUM_CLASSES + c] = (
                                    acc[j * _NUM_CLASSES + c] + xs[j][i] * ws[i]
                                )
                return tuple(acc)

            acc = list(lax.fori_loop(0, _L_CHUNK, l_body, tuple(acc)))

        lane = lax.iota(jnp.int32, 16)
        for j in range(_B_BLK):
            svec = jnp.zeros((16,), jnp.float32)
            for c in range(_NUM_CLASSES):
                s = jnp.broadcast_to(jnp.sum(acc[j * _NUM_CLASSES + c]), (16,))
                svec = jnp.where(lane == c, s, svec)
            out_v[g * _B_BLK + j, :] = svec
        return carry

    lax.fori_loop(0, _N_GROUPS, group_body, 0)
    pltpu.sync_copy(out_v, out.at[pl.ds(b0, _B_PER_TILE)])


def _sc_logits(table_pad, batchr, w_packed):
    mesh = plsc.VectorSubcoreMesh(
        core_axis_name="c", subcore_axis_name="s",
        num_cores=_NC, num_subcores=_NS,
    )
    call = functools.partial(
        pl.kernel,
        out_type=jax.ShapeDtypeStruct((_BATCH, 16), jnp.float32),
        mesh=mesh,
        scratch_types=[
            pltpu.VMEM((_NUM_CLASSES, _MAX_LEN, _NGRP, 32), jnp.bfloat16),
            pltpu.VMEM((_B_BLK, _L_CHUNK, _DPAD), jnp.float32),
            pltpu.VMEM((_B_BLK, _N_CHUNK, _L_CHUNK), jnp.int32),
            pltpu.VMEM((_B_PER_TILE, 16), jnp.float32),
            pltpu.SemaphoreType.DMA,
        ],
        compiler_params=pltpu.CompilerParams(
            needs_layout_passes=False, use_tc_tiling_on_sc=False),
    )(_sc_body)
    return call(table_pad, batchr, w_packed)


def _tc_logsoftmax_body(x_ref, b_ref, o_ref):
    x = x_ref[...][:, : _NUM_CLASSES] + b_ref[...]
    m = jnp.max(x, axis=-1, keepdims=True)
    e = jnp.exp(x - m)
    o_ref[...] = (x - m) - jnp.log(jnp.sum(e, axis=-1, keepdims=True))


def _pack_weights(fc_w):
    w3 = fc_w.reshape(_NUM_CLASSES, _MAX_LEN, _EMBED_DIM)
    w_pad = jnp.pad(w3, ((0, 0), (0, 0), (0, _DPAD - _EMBED_DIM)))
    wr = w_pad.reshape(_NUM_CLASSES, _MAX_LEN, _KV, 16)
    zero = jnp.zeros_like(wr[:, :, 0])
    a_half = jnp.stack([wr[:, :, 0], wr[:, :, 2], wr[:, :, 4], wr[:, :, 6]], axis=2)
    b_half = jnp.stack([wr[:, :, 1], wr[:, :, 3], wr[:, :, 5], zero], axis=2)
    inter = jnp.stack([a_half, b_half], axis=-1)
    return inter.reshape(_NUM_CLASSES, _MAX_LEN, _NGRP, 32).astype(jnp.bfloat16)


def kernel(batch, emb_table, fc_w, fc_b):
    table_pad = jnp.pad(emb_table, ((0, 0), (0, _DPAD - _EMBED_DIM)))
    batchr = batch.astype(jnp.int32).reshape(_BATCH, _N_CHUNK, _L_CHUNK)
    w_packed = _pack_weights(fc_w)

    logits = _sc_logits(table_pad, batchr, w_packed)

    return pl.pallas_call(
        _tc_logsoftmax_body,
        out_shape=jax.ShapeDtypeStruct((_BATCH, _NUM_CLASSES), jnp.float32),
    )(logits, fc_b.reshape(1, _NUM_CLASSES))
